# Initial kernel scaffold; baseline (speedup 1.0000x reference)
import jax
import jax.numpy as jnp
from jax.experimental import pallas as pl


def kernel(features, edge_index, noise, W0, W1, W2, W3, W4, W5, W6, b0, b1, b2, b3, b4, b5, b6):
    raise NotImplementedError("write your pallas kernel here")


# speedup vs baseline: 1.0404x; 1.0404x over previous
import functools

import jax
import jax.numpy as jnp
from jax import lax
from jax.experimental import pallas as pl
from jax.experimental.pallas import tpu as pltpu

N = 10000
E = 160000
N_PAD = 10240
E_PAD = 163840
BN = 1000


def _scales_body(in2_ref, out2_ref, inv_ref, rin_ref, rout_ref):
    ind = jnp.maximum(in2_ref[0, :, 0] + in2_ref[1, :, 0], 1.0)
    outd = jnp.maximum(out2_ref[0, :, 0] + out2_ref[1, :, 0], 1.0)
    inv_ref[...] = (1.0 / ind)[:, None]
    rin_ref[...] = jax.lax.rsqrt(ind)[:, None]
    rout_ref[...] = jax.lax.rsqrt(outd)[:, None]


def _deg_scales(in2, out2):
    grid = N // BN
    spec = pl.BlockSpec((2, BN, 16), lambda i: (0, i, 0))
    ospec = pl.BlockSpec((BN, 1), lambda i: (i, 0))
    return pl.pallas_call(
        _scales_body,
        grid=(grid,),
        in_specs=[spec, spec],
        out_specs=[ospec, ospec, ospec],
        out_shape=[jax.ShapeDtypeStruct((N, 1), jnp.float32)] * 3,
    )(in2, out2)


def _lin_body(nparts, relu, post, x_ref, w_ref, b_ref, spre_ref, spost_ref, o_ref):
    x = x_ref[0]
    for p in range(1, nparts):
        x = x + x_ref[p]
    t = jnp.dot(x, w_ref[...], preferred_element_type=jnp.float32)
    t = spre_ref[...] * t + b_ref[...][None, :]
    if relu:
        t = jnp.maximum(t, 0.0)
    if post:
        t = spost_ref[...] * t
    o_ref[...] = t


def _linear(x_parts, w, b, s_pre, s_post, relu, out_rows):
    P, _, fi = x_parts.shape
    fo = w.shape[1]
    grid = N // BN
    return pl.pallas_call(
        functools.partial(_lin_body, P, relu, s_post is not None),
        grid=(grid,),
        in_specs=[
            pl.BlockSpec((P, BN, fi), lambda i: (0, i, 0)),
            pl.BlockSpec((fi, fo), lambda i: (0, 0)),
            pl.BlockSpec((fo,), lambda i: (0,)),
            pl.BlockSpec((BN, 1), lambda i: (i, 0)),
            pl.BlockSpec((BN, 1), lambda i: (i, 0)),
        ],
        out_specs=pl.BlockSpec((BN, fo), lambda i: (i, 0)),
        out_shape=jax.ShapeDtypeStruct((out_rows, fo), jnp.float32),
    )(x_parts, w, b, s_pre, s_pre if s_post is None else s_post)


def _lin2_body(nparts, x_ref, w_ref, b_ref, wc_ref, rin_ref, rout_ref, o_ref):
    x = x_ref[0]
    for p in range(1, nparts):
        x = x + x_ref[p]
    t = jnp.dot(x, w_ref[...], preferred_element_type=jnp.float32)
    t = jnp.maximum(rin_ref[...] * t + b_ref[...][None, :], 0.0)
    o_ref[...] = jnp.dot(rout_ref[...] * t, wc_ref[...],
                         preferred_element_type=jnp.float32)


def _linear2(x_parts, w, b, wcat, rin, rout, out_rows):
    P, _, fi = x_parts.shape
    fo = w.shape[1]
    fc = wcat.shape[1]
    grid = N // BN
    return pl.pallas_call(
        functools.partial(_lin2_body, P),
        grid=(grid,),
        in_specs=[
            pl.BlockSpec((P, BN, fi), lambda i: (0, i, 0)),
            pl.BlockSpec((fi, fo), lambda i: (0, 0)),
            pl.BlockSpec((fo,), lambda i: (0,)),
            pl.BlockSpec((fo, fc), lambda i: (0, 0)),
            pl.BlockSpec((BN, 1), lambda i: (i, 0)),
            pl.BlockSpec((BN, 1), lambda i: (i, 0)),
        ],
        out_specs=pl.BlockSpec((BN, fc), lambda i: (i, 0)),
        out_shape=jax.ShapeDtypeStruct((out_rows, fc), jnp.float32),
    )(x_parts, w, b, wcat, rin, rout)


def _head_body(nparts, x_ref, rin_ref, bc_ref, noise_ref, mean_ref, ls_ref, z_ref):
    x = x_ref[0]
    for p in range(1, nparts):
        x = x + x_ref[p]
    ml = rin_ref[...] * x + bc_ref[...][None, :]
    mean = ml[:, :32]
    log_std = ml[:, 32:]
    mean_ref[...] = mean
    ls_ref[...] = log_std
    z_ref[...] = mean + noise_ref[...] * jnp.exp(log_std * 0.5)


def _head(x_parts, rin, bcat, noise):
    P = x_parts.shape[0]
    grid = N // BN
    o32 = pl.BlockSpec((BN, 32), lambda i: (i, 0))
    return pl.pallas_call(
        functools.partial(_head_body, P),
        grid=(grid,),
        in_specs=[
            pl.BlockSpec((P, BN, 64), lambda i: (0, i, 0)),
            pl.BlockSpec((BN, 1), lambda i: (i, 0)),
            pl.BlockSpec((64,), lambda i: (0,)),
            o32,
        ],
        out_specs=[o32, o32, o32],
        out_shape=[jax.ShapeDtypeStruct((N, 32), jnp.float32)] * 3,
    )(x_parts, rin, bcat, noise)


def _dec_body(za_ref, zb_ref, o_ref):
    t = jnp.dot(za_ref[...], zb_ref[...].T, preferred_element_type=jnp.float32)
    o_ref[...] = jax.nn.sigmoid(t)


def _decoder(z):
    bc = 1280
    return pl.pallas_call(
        _dec_body,
        grid=(N // BN, pl.cdiv(N, bc)),
        in_specs=[
            pl.BlockSpec((BN, 32), lambda i, j: (i, 0)),
            pl.BlockSpec((bc, 32), lambda i, j: (j, 0)),
        ],
        out_specs=pl.BlockSpec((BN, bc), lambda i, j: (i, j)),
        out_shape=jax.ShapeDtypeStruct((N, N), jnp.float32),
    )(z, z)


def _agg_parts_jax(x, src, dst, width):
    agg = jnp.zeros((N, width), jnp.float32).at[dst].add(x[:N, :][src])
    return jnp.stack([agg, jnp.zeros_like(agg)])


def _deg_parts_jax(src, dst):
    ind = jnp.bincount(dst, length=N).astype(jnp.float32)
    outd = jnp.bincount(src, length=N).astype(jnp.float32)

    def expand(d):
        full = jnp.zeros((N_PAD, 16), jnp.float32).at[:N, 0].set(d)
        return jnp.stack([full, jnp.zeros_like(full)])

    return expand(ind), expand(outd)


def kernel(features, edge_index, noise, W0, W1, W2, W3, W4, W5, W6,
           b0, b1, b2, b3, b4, b5, b6):
    src = edge_index[0]
    dst = edge_index[1]

    in2, out2 = _deg_parts_jax(src, dst)
    inv_in, rin, rout = _deg_scales(in2, out2)

    wcat = jnp.concatenate([W5, W6], axis=1)
    bcat = jnp.concatenate([b5, b6], axis=0)

    a0 = _agg_parts_jax(features, src, dst, 128)
    xs1 = _linear(a0, W0, b0, inv_in, rout, True, N_PAD)
    a1 = _agg_parts_jax(xs1, src, dst, 128)
    xs2 = _linear(a1, W1, b1, rin, rout, True, N_PAD)
    a2 = _agg_parts_jax(xs2, src, dst, 128)
    xs3 = _linear(a2, W2, b2, rin, rout, True, N_PAD)
    a3 = _agg_parts_jax(xs3, src, dst, 192)
    xs4 = _linear(a3, W3, b3, rin, rout, True, N_PAD)
    a4 = _agg_parts_jax(xs4, src, dst, 256)
    y = _linear2(a4, W4, b4, wcat, rin, rout, N_PAD)
    a5 = _agg_parts_jax(y, src, dst, 64)
    mean, log_std, z = _head(a5, rin, bcat, noise)

    adj_rec = _decoder(z)
    return (adj_rec, mean, log_std)


# speedup vs baseline: 3.4400x; 3.3064x over previous
import functools

import jax
import jax.numpy as jnp
from jax import lax
from jax.experimental import pallas as pl
from jax.experimental.pallas import tpu as pltpu
from jax.experimental.pallas import tpu_sc as plsc

N = 10000
E = 160000
N_PAD = 10240
E_PAD = 163840
BN = 1000

BLK = 128
NBLK_A = 80
NBLK_D = 40
RPT = N_PAD // 16
ZR = 64
NBUF = 4


def _scales_body(in2_ref, out2_ref, inv_ref, rin_ref, rout_ref):
    ind = jnp.maximum(in2_ref[0, :, 0] + in2_ref[1, :, 0], 1.0)
    outd = jnp.maximum(out2_ref[0, :, 0] + out2_ref[1, :, 0], 1.0)
    inv_ref[...] = (1.0 / ind)[:, None]
    rin_ref[...] = jax.lax.rsqrt(ind)[:, None]
    rout_ref[...] = jax.lax.rsqrt(outd)[:, None]


def _deg_scales(in2, out2):
    spec = pl.BlockSpec((2, BN, 16), lambda i: (0, i, 0))
    ospec = pl.BlockSpec((BN, 1), lambda i: (i, 0))
    return pl.pallas_call(
        _scales_body,
        grid=(N // BN,),
        in_specs=[spec, spec],
        out_specs=[ospec, ospec, ospec],
        out_shape=[jax.ShapeDtypeStruct((N, 1), jnp.float32)] * 3,
    )(in2, out2)


def _lin_body(ngroups, relu, post, *refs):
    x_refs = refs[:ngroups]
    w_ref, b_ref, spre_ref, spost_ref = refs[ngroups:ngroups + 4]
    o_refs = refs[ngroups + 4:]
    t = None
    off = 0
    for x_ref in x_refs:
        fh = x_ref.shape[2]
        for h in range(2):
            part = jnp.dot(x_ref[h], w_ref[off:off + fh, :],
                           preferred_element_type=jnp.float32)
            t = part if t is None else t + part
            off += fh
    t = spre_ref[...] * t + b_ref[...][None, :]
    if relu:
        t = jnp.maximum(t, 0.0)
    if post:
        t = spost_ref[...] * t
    off = 0
    for o_ref in o_refs:
        hw = o_ref.shape[2]
        o_ref[0] = t[:, off:off + hw]
        o_ref[1] = t[:, off + hw:off + 2 * hw]
        off += 2 * hw


def _linear(x_groups, w, b, s_pre, s_post, relu, out_halves):
    fo = w.shape[1]
    fi = w.shape[0]
    return pl.pallas_call(
        functools.partial(_lin_body, len(x_groups), relu, s_post is not None),
        grid=(N // BN,),
        in_specs=[pl.BlockSpec((2, BN, g.shape[2]), lambda i: (0, i, 0))
                  for g in x_groups] + [
            pl.BlockSpec((fi, fo), lambda i: (0, 0)),
            pl.BlockSpec((fo,), lambda i: (0,)),
            pl.BlockSpec((BN, 1), lambda i: (i, 0)),
            pl.BlockSpec((BN, 1), lambda i: (i, 0)),
        ],
        out_specs=[pl.BlockSpec((2, BN, hw), lambda i: (0, i, 0))
                   for hw in out_halves],
        out_shape=[jax.ShapeDtypeStruct((2, N_PAD, hw), jnp.float32)
                   for hw in out_halves],
    )(*x_groups, w, b, s_pre, s_pre if s_post is None else s_post)


def _lin2_body(xa_ref, xb_ref, w_ref, b_ref, wc_ref, rin_ref, rout_ref,
               o_ref):
    t = (jnp.dot(xa_ref[0], w_ref[0:64, :], preferred_element_type=jnp.float32)
         + jnp.dot(xa_ref[1], w_ref[64:128, :],
                   preferred_element_type=jnp.float32)
         + jnp.dot(xb_ref[0], w_ref[128:192, :],
                   preferred_element_type=jnp.float32)
         + jnp.dot(xb_ref[1], w_ref[192:256, :],
                   preferred_element_type=jnp.float32))
    t = jnp.maximum(rin_ref[...] * t + b_ref[...][None, :], 0.0)
    y = jnp.dot(rout_ref[...] * t, wc_ref[...],
                preferred_element_type=jnp.float32)
    o_ref[0] = y[:, :32]
    o_ref[1] = y[:, 32:]


def _linear2(xa, xb, w, b, wcat, rin, rout):
    fo = w.shape[1]
    fc = wcat.shape[1]
    xspec = pl.BlockSpec((2, BN, 64), lambda i: (0, i, 0))
    return pl.pallas_call(
        _lin2_body,
        grid=(N // BN,),
        in_specs=[
            xspec,
            xspec,
            pl.BlockSpec((256, fo), lambda i: (0, 0)),
            pl.BlockSpec((fo,), lambda i: (0,)),
            pl.BlockSpec((fo, fc), lambda i: (0, 0)),
            pl.BlockSpec((BN, 1), lambda i: (i, 0)),
            pl.BlockSpec((BN, 1), lambda i: (i, 0)),
        ],
        out_specs=pl.BlockSpec((2, BN, 32), lambda i: (0, i, 0)),
        out_shape=jax.ShapeDtypeStruct((2, N_PAD, 32), jnp.float32),
    )(xa, xb, w, b, wcat, rin, rout)


def _head_body(x_ref, rin_ref, bc_ref, noise_ref, mean_ref, ls_ref, z_ref):
    mean = rin_ref[...] * x_ref[0] + bc_ref[...][None, :32]
    log_std = rin_ref[...] * x_ref[1] + bc_ref[...][None, 32:]
    mean_ref[...] = mean
    ls_ref[...] = log_std
    z_ref[...] = mean + noise_ref[...] * jnp.exp(log_std * 0.5)


def _head(x2, rin, bcat, noise):
    o32 = pl.BlockSpec((BN, 32), lambda i: (i, 0))
    return pl.pallas_call(
        _head_body,
        grid=(N // BN,),
        in_specs=[
            pl.BlockSpec((2, BN, 32), lambda i: (0, i, 0)),
            pl.BlockSpec((BN, 1), lambda i: (i, 0)),
            pl.BlockSpec((64,), lambda i: (0,)),
            o32,
        ],
        out_specs=[o32, o32, o32],
        out_shape=[jax.ShapeDtypeStruct((N, 32), jnp.float32)] * 3,
    )(x2, rin, bcat, noise)


def _dec_body(za_ref, zb_ref, o_ref):
    t = jnp.dot(za_ref[...], zb_ref[...].T, preferred_element_type=jnp.float32)
    o_ref[...] = jax.nn.sigmoid(t)


def _decoder(z):
    bc = 1280
    return pl.pallas_call(
        _dec_body,
        grid=(N // BN, pl.cdiv(N, bc)),
        in_specs=[
            pl.BlockSpec((BN, 32), lambda i, j: (i, 0)),
            pl.BlockSpec((bc, 32), lambda i, j: (j, 0)),
        ],
        out_specs=pl.BlockSpec((BN, bc), lambda i, j: (i, j)),
        out_shape=jax.ShapeDtypeStruct((N, N), jnp.float32),
    )(z, z)


def _zero_rows(buf, rows, width):
    @pl.loop(0, rows)
    def _(r):
        for q in range(width // 16):
            buf[r, pl.ds(q * 16, 16)] = jnp.zeros((16,), jnp.float32)


def _make_agg_sc(FH):
    mesh = plsc.VectorSubcoreMesh(core_axis_name="c", subcore_axis_name="s")
    scratch = [
        pltpu.VMEM((NBLK_A, BLK), jnp.int32),
        pltpu.VMEM((NBLK_A, BLK), jnp.int32),
        pltpu.VMEM((NBUF, BLK, FH), jnp.float32),
        pltpu.VMEM((ZR, FH), jnp.float32),
        pltpu.VMEM_SHARED((N_PAD, FH), jnp.float32),
    ] + [pltpu.SemaphoreType.DMA] * (2 * NBUF + 1)

    @functools.partial(
        pl.kernel,
        out_type=jax.ShapeDtypeStruct((2, N_PAD, FH), jnp.float32),
        mesh=mesh,
        scratch_types=scratch,
        compiler_params=pltpu.CompilerParams(use_tc_tiling_on_sc=False),
    )
    def k(x_hbm, src_hbm, dst_hbm, out_hbm, src_v, dst_v, bufs, zbuf, agg_sh,
          *sems):
        gsem = sems[:NBUF]
        ssem = sems[NBUF:2 * NBUF]
        msem = sems[2 * NBUF]
        c = lax.axis_index("c")
        s = lax.axis_index("s")
        pltpu.async_copy(src_hbm.at[s], src_v, msem).wait()
        pltpu.async_copy(dst_hbm.at[s], dst_v, msem).wait()
        xc = x_hbm.at[c]
        _zero_rows(zbuf, ZR, FH)
        for q in range(RPT // ZR):
            pltpu.async_copy(
                zbuf, agg_sh.at[pl.ds(s * RPT + q * ZR, ZR)], msem).wait()
        plsc.subcore_barrier()
        for u in range(NBUF):
            pltpu.make_async_copy(
                xc.at[src_v.at[u]], bufs.at[u], gsem[u]).start()

        @pl.loop(0, NBLK_A, step=NBUF)
        def _(j0):
            for u in range(NBUF):
                j = j0 + u
                pltpu.make_async_copy(
                    xc.at[src_v.at[j]], bufs.at[u], gsem[u]).wait()
                pltpu.make_async_copy(
                    bufs.at[u], agg_sh.at[dst_v.at[j]], ssem[u]).start(add=True)
                jn = j + NBUF

                @pl.when(jn < NBLK_A)
                def _():
                    pltpu.make_async_copy(
                        bufs.at[u], agg_sh.at[dst_v.at[j]], ssem[u]).wait()
                    pltpu.make_async_copy(
                        xc.at[src_v.at[jn]], bufs.at[u], gsem[u]).start()

        for u in range(NBUF):
            pltpu.make_async_copy(
                bufs.at[u], agg_sh.at[dst_v.at[NBLK_A - NBUF + u]],
                ssem[u]).wait()
        plsc.subcore_barrier()
        pltpu.async_copy(
            agg_sh.at[pl.ds(s * RPT, RPT)],
            out_hbm.at[c].at[pl.ds(s * RPT, RPT)], msem).wait()

    return k


_AGG_SC = {FH: _make_agg_sc(FH) for FH in (32, 48, 64)}


def _make_deg_sc():
    mesh = plsc.VectorSubcoreMesh(core_axis_name="c", subcore_axis_name="s")
    scratch = [
        pltpu.VMEM((NBLK_D, BLK), jnp.int32),
        pltpu.VMEM((NBLK_D, BLK), jnp.int32),
        pltpu.VMEM((BLK, 16), jnp.float32),
        pltpu.VMEM((ZR, 16), jnp.float32),
        pltpu.VMEM_SHARED((N_PAD, 16), jnp.float32),
        pltpu.VMEM_SHARED((N_PAD, 16), jnp.float32),
    ] + [pltpu.SemaphoreType.DMA] * 3

    @functools.partial(
        pl.kernel,
        out_type=[jax.ShapeDtypeStruct((2, N_PAD, 16), jnp.float32)] * 2,
        mesh=mesh,
        scratch_types=scratch,
        compiler_params=pltpu.CompilerParams(use_tc_tiling_on_sc=False),
    )
    def k(src_hbm, dst_hbm, ind_hbm, outd_hbm, src_v, dst_v, ones_v, zbuf,
          in_sh, out_sh, sA, sB, msem):
        c = lax.axis_index("c")
        s = lax.axis_index("s")
        wid = c * 16 + s
        pltpu.async_copy(src_hbm.at[wid], src_v, msem).wait()
        pltpu.async_copy(dst_hbm.at[wid], dst_v, msem).wait()

        @pl.loop(0, BLK)
        def _(r):
            ones_v[r, pl.ds(0, 16)] = jnp.ones((16,), jnp.float32)

        _zero_rows(zbuf, ZR, 16)
        for q in range(RPT // ZR):
            pltpu.async_copy(
                zbuf, in_sh.at[pl.ds(s * RPT + q * ZR, ZR)], msem).wait()
            pltpu.async_copy(
                zbuf, out_sh.at[pl.ds(s * RPT + q * ZR, ZR)], msem).wait()
        plsc.subcore_barrier()

        @pl.loop(0, NBLK_D)
        def _(j):
            pltpu.make_async_copy(
                ones_v, in_sh.at[dst_v.at[j]], sA).start(add=True)
            pltpu.make_async_copy(
                ones_v, out_sh.at[src_v.at[j]], sB).start(add=True)

        @pl.loop(0, NBLK_D)
        def _(j):
            pltpu.make_async_copy(ones_v, in_sh.at[dst_v.at[0]], sA).wait()
            pltpu.make_async_copy(ones_v, out_sh.at[src_v.at[0]], sB).wait()

        plsc.subcore_barrier()
        pltpu.async_copy(
            in_sh.at[pl.ds(s * RPT, RPT)],
            ind_hbm.at[c].at[pl.ds(s * RPT, RPT)], msem).wait()
        pltpu.async_copy(
            out_sh.at[pl.ds(s * RPT, RPT)],
            outd_hbm.at[c].at[pl.ds(s * RPT, RPT)], msem).wait()

    return k


_DEG_SC = _make_deg_sc()


def kernel(features, edge_index, noise, W0, W1, W2, W3, W4, W5, W6,
           b0, b1, b2, b3, b4, b5, b6):
    pad = jnp.full((E_PAD - E,), N, jnp.int32)
    srcp = jnp.concatenate([edge_index[0], pad])
    dstp = jnp.concatenate([edge_index[1], pad])
    src_d = srcp.reshape(32, NBLK_D, BLK)
    dst_d = dstp.reshape(32, NBLK_D, BLK)
    src_a = srcp.reshape(16, NBLK_A, BLK)
    dst_a = dstp.reshape(16, NBLK_A, BLK)

    in2, out2 = _DEG_SC(src_d, dst_d)
    inv_in, rin, rout = _deg_scales(in2, out2)

    wcat = jnp.concatenate([W5, W6], axis=1)
    bcat = jnp.concatenate([b5, b6], axis=0)
    feat2 = jnp.pad(features, ((0, N_PAD - N), (0, 0)))
    feat2 = jnp.stack([feat2[:, :64], feat2[:, 64:]])

    a0 = _AGG_SC[64](feat2, src_a, dst_a)
    (xs1,) = _linear((a0,), W0, b0, inv_in, rout, True, (64,))
    a1 = _AGG_SC[64](xs1, src_a, dst_a)
    (xs2,) = _linear((a1,), W1, b1, rin, rout, True, (64,))
    a2 = _AGG_SC[64](xs2, src_a, dst_a)
    xs3a, xs3b = _linear((a2,), W2, b2, rin, rout, True, (48, 48))
    a3a = _AGG_SC[48](xs3a, src_a, dst_a)
    a3b = _AGG_SC[48](xs3b, src_a, dst_a)
    xs4a, xs4b = _linear((a3a, a3b), W3, b3, rin, rout, True, (64, 64))
    a4a = _AGG_SC[64](xs4a, src_a, dst_a)
    a4b = _AGG_SC[64](xs4b, src_a, dst_a)
    y = _linear2(a4a, a4b, W4, b4, wcat, rin, rout)
    a5 = _AGG_SC[32](y, src_a, dst_a)
    mean, log_std, z = _head(a5, rin, bcat, noise)

    adj_rec = _decoder(z)
    return (adj_rec, mean, log_std)
